# baseline (speedup 1.0000x reference)
import math

import jax
import jax.numpy as jnp
from jax.experimental import pallas as pl

B = 8; QL = 1; D = 2048; H = 16; DH = 128; NKV = 8; GROUPS = 2
KV = 2048; PAST = KV - 1; HIST = 64; POOL = 16; PLEN = KV // POOL
TOPK = 1024; SINK = 64; LOCAL = 64
NROW = B * H
NBLK = PLEN
NSEL = TOPK // POOL
SCALE = 1.0 / math.sqrt(DH)
IMG = HIST * PLEN


def _rope(y, cosv, sinv, nheads):
    parts = []
    for h in range(nheads):
        sl = y[:, h * DH:(h + 1) * DH]
        rot = jnp.concatenate([-sl[:, DH // 2:], sl[:, :DH // 2]], axis=1)
        parts.append(sl * cosv + rot * sinv)
    return jnp.concatenate(parts, axis=1)


def _qkv_kernel(hs_ref, wq_ref, wk_ref, wv_ref, cos_ref, sin_ref,
                q_ref, k_ref, v_ref):
    hs = hs_ref[...]
    cosv = cos_ref[...]
    sinv = sin_ref[...]
    dn = (((1,), (1,)), ((), ()))
    yq = jax.lax.dot_general(hs, wq_ref[...], dn,
                             preferred_element_type=jnp.float32)
    yk = jax.lax.dot_general(hs, wk_ref[...], dn,
                             preferred_element_type=jnp.float32)
    yv = jax.lax.dot_general(hs, wv_ref[...], dn,
                             preferred_element_type=jnp.float32)
    q_ref[...] = _rope(yq, cosv, sinv, H)
    k_ref[...] = _rope(yk, cosv, sinv, NKV)
    v_ref[...] = yv


def _cnn_rows(x, ns, w1, b1, w2, b2, w3, b3):
    n = ns * IMG
    gi = jax.lax.broadcasted_iota(jnp.int32, (1, n), 1)
    wpos = gi % PLEN
    hpos = (gi // PLEN) % HIST
    taps = []
    for di in (-1, 0, 1):
        for dj in (-1, 0, 1):
            off = di * PLEN + dj
            valid = ((hpos + di >= 0) & (hpos + di < HIST)
                     & (wpos + dj >= 0) & (wpos + dj < PLEN))
            taps.append((off, valid))

    def shifts(img):
        outs = []
        for off, valid in taps:
            r = img if off == 0 else jnp.roll(img, -off, axis=1)
            outs.append(jnp.where(valid, r, 0.0))
        return outs

    dn = (((1,), (0,)), ((), ()))
    pat1 = jnp.concatenate(shifts(x), axis=0)
    y1 = jax.lax.dot_general(w1, pat1, dn,
                             preferred_element_type=jnp.float32)
    y1 = jnp.maximum(y1 + b1, 0.0)
    pat2 = jnp.concatenate(shifts(y1), axis=0)
    y2 = jax.lax.dot_general(w2, pat2, dn,
                             preferred_element_type=jnp.float32)
    y2 = jnp.maximum(y2 + b2, 0.0)
    cols = []
    for s in range(ns):
        acc = jnp.zeros((32, PLEN), dtype=jnp.float32)
        base = s * IMG
        for h in range(HIST):
            acc = acc + y2[:, base + h * PLEN:base + (h + 1) * PLEN]
        cols.append(acc * (1.0 / HIST))
    m = jnp.concatenate(cols, axis=1)
    return jax.lax.dot_general(w3, m, dn,
                               preferred_element_type=jnp.float32) + b3


def _select_rows(tsp, ns):
    lane = jax.lax.broadcasted_iota(jnp.int32, (1, NBLK), 1)
    rows = []
    for s in range(ns):
        t = tsp[:, s * NBLK:(s + 1) * NBLK]
        rank = jnp.zeros((1, NBLK), dtype=jnp.int32)
        for j in range(NBLK):
            vj = t[:, j:j + 1]
            cond = (vj > t) | ((vj == t) & (j < lane))
            rank = rank + cond.astype(jnp.int32)
        sel = ((rank < NSEL) | (lane < SINK // POOL)
               | (lane >= NBLK - LOCAL // POOL))
        rows.append(jnp.where(sel, 0.0, -1e9).astype(jnp.float32))
    return jnp.concatenate(rows, axis=0)


def _attn_kernel(ah_ref, w1_ref, b1_ref, w2_ref, b2_ref, w3_ref, b3_ref,
                 q_ref, kn_ref, vn_ref, kp_ref, vp_ref, o_ref):
    tsp = _cnn_rows(ah_ref[0], GROUPS, w1_ref[...], b1_ref[...], w2_ref[...],
                    b2_ref[...], w3_ref[...], b3_ref[...])
    mask_blk = _select_rows(tsp, GROUPS)
    pos = jax.lax.broadcasted_iota(jnp.int32, (NBLK, KV), 1) // POOL
    blk = jax.lax.broadcasted_iota(jnp.int32, (NBLK, KV), 0)
    expand = (pos == blk).astype(jnp.float32)
    mask = jax.lax.dot_general(mask_blk, expand, (((1,), (0,)), ((), ())),
                               preferred_element_type=jnp.float32)

    qh = q_ref[0]
    kp = kp_ref[0, 0]
    vp = vp_ref[0, 0]
    s_p = jax.lax.dot_general(qh, kp, (((1,), (1,)), ((), ())),
                              preferred_element_type=jnp.float32)
    s_n = jax.lax.dot_general(qh, kn_ref[0], (((1,), (1,)), ((), ())),
                              preferred_element_type=jnp.float32)
    logits = jnp.concatenate([s_p, s_n], axis=1) * SCALE + mask
    mx = jnp.max(logits, axis=1, keepdims=True)
    e = jnp.exp(logits - mx)
    den = jnp.sum(e, axis=1, keepdims=True)
    p = e / den
    o = jax.lax.dot_general(p[:, :PAST], vp, (((1,), (0,)), ((), ())),
                            preferred_element_type=jnp.float32)
    o = o + p[:, PAST:] * vn_ref[0]
    o_ref[0] = o


def _outproj_kernel(x_ref, w_ref, o_ref):
    o_ref[...] = jax.lax.dot_general(
        x_ref[...], w_ref[...], (((1,), (1,)), ((), ())),
        preferred_element_type=jnp.float32)


def kernel(hidden_states, past_key, past_value, attn_history, cos, sin,
           wq, wk, wv, wo, c1w, c1b, c2w, c2b, c3w, c3b):
    f32 = jnp.float32
    hs = hidden_states.reshape(B, D)
    cosv = cos[0, 0].reshape(1, DH)
    sinv = sin[0, 0].reshape(1, DH)

    q_flat, k_flat, v_flat = pl.pallas_call(
        _qkv_kernel,
        out_shape=(jax.ShapeDtypeStruct((B, H * DH), f32),
                   jax.ShapeDtypeStruct((B, NKV * DH), f32),
                   jax.ShapeDtypeStruct((B, NKV * DH), f32)),
    )(hs, wq, wk, wv, cosv, sinv)

    ah = attn_history.reshape(B * NKV, 1, GROUPS * IMG)
    w1r = c1w.reshape(16, 9)
    w2r = c2w.transpose(0, 2, 3, 1).reshape(32, 144)
    w3r = c3w[:, :, 0]

    q3 = q_flat.reshape(B * NKV, GROUPS, DH)
    kn = k_flat.reshape(B * NKV, 1, DH)
    vn = v_flat.reshape(B * NKV, 1, DH)

    attn_out = pl.pallas_call(
        _attn_kernel,
        grid=(B * NKV,),
        in_specs=[
            pl.BlockSpec((1, 1, GROUPS * IMG), lambda i: (i, 0, 0)),
            pl.BlockSpec((16, 9), lambda i: (0, 0)),
            pl.BlockSpec((16, 1), lambda i: (0, 0)),
            pl.BlockSpec((32, 144), lambda i: (0, 0)),
            pl.BlockSpec((32, 1), lambda i: (0, 0)),
            pl.BlockSpec((1, 32), lambda i: (0, 0)),
            pl.BlockSpec((1, 1), lambda i: (0, 0)),
            pl.BlockSpec((1, GROUPS, DH), lambda i: (i, 0, 0)),
            pl.BlockSpec((1, 1, DH), lambda i: (i, 0, 0)),
            pl.BlockSpec((1, 1, DH), lambda i: (i, 0, 0)),
            pl.BlockSpec((1, 1, PAST, DH), lambda i: (i // NKV, i % NKV, 0, 0)),
            pl.BlockSpec((1, 1, PAST, DH), lambda i: (i // NKV, i % NKV, 0, 0)),
        ],
        out_specs=pl.BlockSpec((1, GROUPS, DH), lambda i: (i, 0, 0)),
        out_shape=jax.ShapeDtypeStruct((B * NKV, GROUPS, DH), f32),
    )(ah, w1r, c1b.reshape(16, 1), w2r, c2b.reshape(32, 1),
      w3r, c3b.reshape(1, 1), q3, kn, vn, past_key, past_value)

    out = pl.pallas_call(
        _outproj_kernel,
        out_shape=jax.ShapeDtypeStruct((B, D), f32),
    )(attn_out.reshape(B, D), wo)
    return out.reshape(B, QL, D)
